# baseline (speedup 1.0000x reference)
import functools

import jax
import jax.numpy as jnp
from jax import lax
from jax.experimental import pallas as pl
from jax.experimental.pallas import tpu as pltpu
from jax.experimental.pallas import tpu_sc as plsc

_B, _S, _H = 1024, 512, 128
_TOK = _B * _S
_NW = 32
_PERW = _TOK // _NW
_CH = 128
_NCH = _PERW // _CH
_G = _H // 16
_LANES = 16
_NPT = _S * 2

_mesh = plsc.VectorSubcoreMesh(
    core_axis_name="c", subcore_axis_name="s", num_cores=2, num_subcores=16
)


def _emb_body(cmb_hbm, wt_hbm, pt_hbm, out_hbm,
              cmb0, cmb1, rows0, rows1, ptr0, ptr1, pt_sh,
              in0, in1, pt0, pt1, os0, os1, is0, is1):
    cmb_v = (cmb0, cmb1)
    rows_v = (rows0, rows1)
    ptr_v = (ptr0, ptr1)
    in_sem = (in0, in1)
    ptg_sem = (pt0, pt1)
    out_sem = (os0, os1)
    ids_sem = (is0, is1)

    sid = lax.axis_index("s")
    wid = sid * 2 + lax.axis_index("c")
    base = wid * _PERW

    @pl.when(sid == 0)
    def _():
        pltpu.sync_copy(pt_hbm, pt_sh)

    plsc.subcore_barrier()

    cbase = wid * _NCH

    def ids_copy(c, k):
        pltpu.async_copy(cmb_hbm.at[cbase + c], cmb_v[k], ids_sem[k])

    def ids_wait(c, k):
        pltpu.make_async_copy(cmb_hbm.at[cbase + c], cmb_v[k], ids_sem[k]).wait()

    def gather_start(k):
        pltpu.async_copy(wt_hbm.at[cmb_v[k].at[0]], rows_v[k], in_sem[k])
        pltpu.async_copy(pt_sh.at[cmb_v[k].at[1]], ptr_v[k], ptg_sem[k])

    def gather_wait(k):
        pltpu.make_async_copy(wt_hbm.at[cmb_v[k].at[0]], rows_v[k], in_sem[k]).wait()
        pltpu.make_async_copy(pt_sh.at[cmb_v[k].at[1]], ptr_v[k], ptg_sem[k]).wait()

    def out_start(c, k):
        off = base + c * _CH
        pltpu.async_copy(rows_v[k], out_hbm.at[pl.ds(off, _CH)], out_sem[k])

    def out_wait(c, k):
        off = base + c * _CH
        pltpu.make_async_copy(rows_v[k], out_hbm.at[pl.ds(off, _CH)], out_sem[k]).wait()

    gather_dnums = lax.GatherDimensionNumbers(
        offset_dims=(), collapsed_slice_dims=(0,), start_index_map=(0,))

    lanes = lax.iota(jnp.int32, _LANES)
    perms = {sh: (lanes ^ sh).reshape(_LANES, 1) for sh in (8, 4, 2, 1)}
    mask_hi = lanes >= 8

    def lperm(v, idx):
        return lax.gather(v, idx, gather_dnums, (1,),
                          mode=lax.GatherScatterMode.PROMISE_IN_BOUNDS)

    def merged_allsum(va, vb):
        va = va + lperm(va, perms[8])
        vb = vb + lperm(vb, perms[8])
        m = jnp.where(mask_hi, vb, va)
        for sh in (4, 2, 1):
            m = m + lperm(m, perms[sh])
        return m

    half = jnp.float32(0.5)
    three_half = jnp.float32(1.5)
    inv_h = jnp.float32(1.0 / _H)
    eps = jnp.float32(1e-12)
    magic = jnp.int32(0x5F3759DF)
    zero16 = jnp.zeros((_LANES,), jnp.float32)

    def compute(ci, k):
        rows = rows_v[k]
        ptr = ptr_v[k]

        @plsc.parallel_loop(0, _CH, 2, unroll=2)
        def tok(i):
            ia = i
            ib = i + 1
            xa = [rows[ia, pl.ds(16 * j, 16)] + ptr[ia, pl.ds(16 * j, 16)]
                  for j in range(_G)]
            xb = [rows[ib, pl.ds(16 * j, 16)] + ptr[ib, pl.ds(16 * j, 16)]
                  for j in range(_G)]

            def tree_sum(vs):
                while len(vs) > 1:
                    vs = [vs[m] + vs[m + 1] for m in range(0, len(vs), 2)]
                return vs[0]

            sa = tree_sum(xa)
            sb = tree_sum(xb)
            qa = tree_sum([x * x for x in xa])
            qb = tree_sum([x * x for x in xb])
            meanv = merged_allsum(sa, sb) * inv_h
            ex2v = merged_allsum(qa, qb) * inv_h
            varv = jnp.maximum(ex2v - meanv * meanv, eps)
            iv = plsc.bitcast(varv, jnp.int32)
            y = plsc.bitcast(magic - (iv >> 1), jnp.float32)
            xh = varv * half
            y = y * (three_half - xh * y * y)
            y = y * (three_half - xh * y * y)
            ms = meanv * y
            yswap = lperm(y, perms[8])
            msswap = lperm(ms, perms[8])
            ya = jnp.where(mask_hi, yswap, y)
            yb = jnp.where(mask_hi, y, yswap)
            msa = jnp.where(mask_hi, msswap, ms)
            msb = jnp.where(mask_hi, ms, msswap)
            for j in range(_G):
                rows[ia, pl.ds(16 * j, 16)] = xa[j] * ya - msa
                rows[ib, pl.ds(16 * j, 16)] = xb[j] * yb - msb

    ids_copy(0, 0)
    ids_copy(1, 1)
    ids_wait(0, 0)
    gather_start(0)

    ids_wait(1, 1)
    gather_start(1)
    gather_wait(0)
    ids_copy(2, 0)
    compute(jnp.int32(0), 0)
    out_start(0, 0)

    ids_wait(2, 0)
    out_wait(0, 0)
    gather_start(0)
    gather_wait(1)
    ids_copy(3, 1)
    compute(jnp.int32(1), 1)
    out_start(1, 1)

    def mid_chunk(ci, k):
        ids_wait(ci + 1, 1 - k)
        out_wait(ci - 1, 1 - k)
        gather_start(1 - k)
        gather_wait(k)
        ids_copy(ci + 2, k)
        compute(ci, k)
        out_start(ci, k)

    def pair(t, carry):
        ci = 2 + 2 * t
        mid_chunk(ci, 0)
        mid_chunk(ci + 1, 1)
        return carry

    lax.fori_loop(0, (_NCH - 4) // 2, pair, 0)

    ci = jnp.int32(_NCH - 2)
    ids_wait(_NCH - 1, 1)
    out_wait(_NCH - 3, 1)
    gather_start(1)
    gather_wait(0)
    compute(ci, 0)
    out_start(ci, 0)

    ci = jnp.int32(_NCH - 1)
    gather_wait(1)
    compute(ci, 1)
    out_start(ci, 1)

    out_wait(_NCH - 2, 0)
    out_wait(_NCH - 1, 1)


_emb = functools.partial(
    pl.kernel,
    out_type=jax.ShapeDtypeStruct((_TOK, _H), jnp.float32),
    mesh=_mesh,
    compiler_params=pltpu.CompilerParams(needs_layout_passes=False),
    scratch_types=[
        pltpu.VMEM((2, _CH), jnp.int32),
        pltpu.VMEM((2, _CH), jnp.int32),
        pltpu.VMEM((_CH, _H), jnp.float32),
        pltpu.VMEM((_CH, _H), jnp.float32),
        pltpu.VMEM((_CH, _H), jnp.float32),
        pltpu.VMEM((_CH, _H), jnp.float32),
        pltpu.VMEM_SHARED((_NPT, _H), jnp.float32),
        pltpu.SemaphoreType.DMA,
        pltpu.SemaphoreType.DMA,
        pltpu.SemaphoreType.DMA,
        pltpu.SemaphoreType.DMA,
        pltpu.SemaphoreType.DMA,
        pltpu.SemaphoreType.DMA,
        pltpu.SemaphoreType.DMA,
        pltpu.SemaphoreType.DMA,
    ],
)(_emb_body)


def kernel(input_ids, token_type_ids, word_table, pos_table, type_table,
           ln_gamma, ln_beta):
    del ln_gamma, ln_beta
    ids = input_ids.reshape(_TOK).astype(jnp.int32)
    ptidx = (jnp.arange(_S, dtype=jnp.int32) * 2)[None, :] + \
        token_type_ids.astype(jnp.int32)
    ptidx = ptidx.reshape(_TOK)
    comb = jnp.stack(
        [ids.reshape(-1, _CH), ptidx.reshape(-1, _CH)], axis=1)
    pt = (pos_table[:, None, :] + type_table[None, :, :]).reshape(_NPT, _H)
    out = _emb(comb, word_table, pt)
    return out.reshape(_B, _S, _H)


# speedup vs baseline: 1.0043x; 1.0043x over previous
import functools

import jax
import jax.numpy as jnp
from jax import lax
from jax.experimental import pallas as pl
from jax.experimental.pallas import tpu as pltpu
from jax.experimental.pallas import tpu_sc as plsc

_B, _S, _H = 1024, 512, 128
_TOK = _B * _S
_NW = 32
_PERW = _TOK // _NW
_CH = 128
_NCH = _PERW // _CH
_G = _H // 16
_LANES = 16
_NPT = _S * 2

_mesh = plsc.VectorSubcoreMesh(
    core_axis_name="c", subcore_axis_name="s", num_cores=2, num_subcores=16
)


def _emb_body(cmb_hbm, wt_hbm, pt_hbm, out_hbm,
              cmb0, cmb1, rows0, rows1, ptr0, ptr1, pt_sh,
              in0, in1, pt0, pt1, os0, os1, is0, is1):
    cmb_v = (cmb0, cmb1)
    rows_v = (rows0, rows1)
    ptr_v = (ptr0, ptr1)
    in_sem = (in0, in1)
    ptg_sem = (pt0, pt1)
    out_sem = (os0, os1)
    ids_sem = (is0, is1)

    sid = lax.axis_index("s")
    wid = sid * 2 + lax.axis_index("c")
    base = wid * _PERW

    @pl.when(sid == 0)
    def _():
        pltpu.sync_copy(pt_hbm, pt_sh)

    plsc.subcore_barrier()

    cbase = wid * _NCH

    def ids_copy(c, k):
        pltpu.async_copy(cmb_hbm.at[cbase + c], cmb_v[k], ids_sem[k])

    def ids_wait(c, k):
        pltpu.make_async_copy(cmb_hbm.at[cbase + c], cmb_v[k], ids_sem[k]).wait()

    def gather_start(k):
        pltpu.async_copy(wt_hbm.at[cmb_v[k].at[0]], rows_v[k], in_sem[k])
        pltpu.async_copy(pt_sh.at[cmb_v[k].at[1]], ptr_v[k], ptg_sem[k])

    def gather_wait(k):
        pltpu.make_async_copy(wt_hbm.at[cmb_v[k].at[0]], rows_v[k], in_sem[k]).wait()
        pltpu.make_async_copy(pt_sh.at[cmb_v[k].at[1]], ptr_v[k], ptg_sem[k]).wait()

    def out_start(c, k):
        off = base + c * _CH
        pltpu.async_copy(rows_v[k], out_hbm.at[pl.ds(off, _CH)], out_sem[k])

    def out_wait(c, k):
        off = base + c * _CH
        pltpu.make_async_copy(rows_v[k], out_hbm.at[pl.ds(off, _CH)], out_sem[k]).wait()

    gather_dnums = lax.GatherDimensionNumbers(
        offset_dims=(), collapsed_slice_dims=(0,), start_index_map=(0,))

    lanes = lax.iota(jnp.int32, _LANES)
    perms = {sh: (lanes ^ sh).reshape(_LANES, 1) for sh in (8, 4, 2, 1)}
    mask_hi = lanes >= 8

    def lperm(v, idx):
        return lax.gather(v, idx, gather_dnums, (1,),
                          mode=lax.GatherScatterMode.PROMISE_IN_BOUNDS)

    def merged_allsum(va, vb):
        va = va + lperm(va, perms[8])
        vb = vb + lperm(vb, perms[8])
        m = jnp.where(mask_hi, vb, va)
        for sh in (4, 2, 1):
            m = m + lperm(m, perms[sh])
        return m

    half = jnp.float32(0.5)
    three_half = jnp.float32(1.5)
    inv_h = jnp.float32(1.0 / _H)
    eps = jnp.float32(1e-12)
    magic = jnp.int32(0x5F3759DF)
    zero16 = jnp.zeros((_LANES,), jnp.float32)

    def compute(ci, k):
        rows = rows_v[k]
        ptr = ptr_v[k]

        @plsc.parallel_loop(0, _CH, 2, unroll=2)
        def tok(i):
            ia = i
            ib = i + 1
            xa = [rows[ia, pl.ds(16 * j, 16)] + ptr[ia, pl.ds(16 * j, 16)]
                  for j in range(_G)]
            xb = [rows[ib, pl.ds(16 * j, 16)] + ptr[ib, pl.ds(16 * j, 16)]
                  for j in range(_G)]

            def tree_sum(vs):
                while len(vs) > 1:
                    vs = [vs[m] + vs[m + 1] for m in range(0, len(vs), 2)]
                return vs[0]

            sa = tree_sum(xa)
            sb = tree_sum(xb)
            qa = tree_sum([x * x for x in xa])
            qb = tree_sum([x * x for x in xb])
            meanv = merged_allsum(sa, sb) * inv_h
            ex2v = merged_allsum(qa, qb) * inv_h
            varv = jnp.maximum(ex2v - meanv * meanv, eps)
            iv = plsc.bitcast(varv, jnp.int32)
            y = plsc.bitcast(magic - (iv >> 1), jnp.float32)
            xh = varv * half
            y = y * (three_half - xh * y * y)
            y = y * (three_half - xh * y * y)
            ms = meanv * y
            yswap = lperm(y, perms[8])
            msswap = lperm(ms, perms[8])
            ya = jnp.where(mask_hi, yswap, y)
            yb = jnp.where(mask_hi, y, yswap)
            msa = jnp.where(mask_hi, msswap, ms)
            msb = jnp.where(mask_hi, ms, msswap)
            for j in range(_G):
                rows[ia, pl.ds(16 * j, 16)] = xa[j] * ya - msa
                rows[ib, pl.ds(16 * j, 16)] = xb[j] * yb - msb

    ids_copy(0, 0)
    ids_copy(1, 1)
    ids_wait(0, 0)
    gather_start(0)

    gather_wait(0)
    ids_wait(1, 1)
    gather_start(1)
    ids_copy(2, 0)
    compute(jnp.int32(0), 0)
    out_start(0, 0)

    gather_wait(1)
    ids_wait(2, 0)
    out_wait(0, 0)
    gather_start(0)
    ids_copy(3, 1)
    compute(jnp.int32(1), 1)
    out_start(1, 1)

    def mid_chunk(ci, k):
        gather_wait(k)
        ids_wait(ci + 1, 1 - k)
        out_wait(ci - 1, 1 - k)
        gather_start(1 - k)
        ids_copy(ci + 2, k)
        compute(ci, k)
        out_start(ci, k)

    def pair(t, carry):
        ci = 2 + 2 * t
        mid_chunk(ci, 0)
        mid_chunk(ci + 1, 1)
        return carry

    lax.fori_loop(0, (_NCH - 4) // 2, pair, 0)

    ci = jnp.int32(_NCH - 2)
    gather_wait(0)
    ids_wait(_NCH - 1, 1)
    out_wait(_NCH - 3, 1)
    gather_start(1)
    compute(ci, 0)
    out_start(ci, 0)

    ci = jnp.int32(_NCH - 1)
    gather_wait(1)
    compute(ci, 1)
    out_start(ci, 1)

    out_wait(_NCH - 2, 0)
    out_wait(_NCH - 1, 1)


_emb = functools.partial(
    pl.kernel,
    out_type=jax.ShapeDtypeStruct((_TOK, _H), jnp.float32),
    mesh=_mesh,
    compiler_params=pltpu.CompilerParams(needs_layout_passes=False),
    scratch_types=[
        pltpu.VMEM((2, _CH), jnp.int32),
        pltpu.VMEM((2, _CH), jnp.int32),
        pltpu.VMEM((_CH, _H), jnp.float32),
        pltpu.VMEM((_CH, _H), jnp.float32),
        pltpu.VMEM((_CH, _H), jnp.float32),
        pltpu.VMEM((_CH, _H), jnp.float32),
        pltpu.VMEM_SHARED((_NPT, _H), jnp.float32),
        pltpu.SemaphoreType.DMA,
        pltpu.SemaphoreType.DMA,
        pltpu.SemaphoreType.DMA,
        pltpu.SemaphoreType.DMA,
        pltpu.SemaphoreType.DMA,
        pltpu.SemaphoreType.DMA,
        pltpu.SemaphoreType.DMA,
        pltpu.SemaphoreType.DMA,
    ],
)(_emb_body)


def kernel(input_ids, token_type_ids, word_table, pos_table, type_table,
           ln_gamma, ln_beta):
    del ln_gamma, ln_beta
    ids = input_ids.reshape(_TOK).astype(jnp.int32)
    ptidx = (jnp.arange(_S, dtype=jnp.int32) * 2)[None, :] + \
        token_type_ids.astype(jnp.int32)
    ptidx = ptidx.reshape(_TOK)
    comb = jnp.stack(
        [ids.reshape(-1, _CH), ptidx.reshape(-1, _CH)], axis=1)
    pt = (pos_table[:, None, :] + type_table[None, :, :]).reshape(_NPT, _H)
    out = _emb(comb, word_table, pt)
    return out.reshape(_B, _S, _H)
